# initial kernel scaffold (unmeasured)
import jax
import jax.numpy as jnp
from jax import lax
from jax.experimental import pallas as pl
from jax.experimental.pallas import tpu as pltpu


def kernel(
    x,
):
    def body(*refs):
        pass

    out_shape = jax.ShapeDtypeStruct(..., jnp.float32)
    return pl.pallas_call(body, out_shape=out_shape)(...)


# baseline (device time: 22395 ns/iter reference)
import jax
import jax.numpy as jnp
from jax import lax
from jax.experimental import pallas as pl
from jax.experimental.pallas import tpu as pltpu

N_DEV = 4


def kernel(x):
    m, n = x.shape

    def body(x_ref, out_ref, sstats, rstats, send_sems, recv_sems):
        my = lax.axis_index("i")

        barrier_sem = pltpu.get_barrier_semaphore()
        for k in range(1, N_DEV):
            pl.semaphore_signal(
                barrier_sem,
                inc=1,
                device_id=((my + k) % N_DEV,),
                device_id_type=pl.DeviceIdType.MESH,
            )
        pl.semaphore_wait(barrier_sem, N_DEV - 1)

        xv = x_ref[:, :]
        lm = jnp.max(xv, axis=1, keepdims=True)
        e = jnp.exp(xv - lm)
        ls = jnp.sum(e, axis=1, keepdims=True)
        out_ref[:, :] = e
        sstats[:, 0:1] = lm
        sstats[:, 1:2] = ls

        sends = []
        for k in range(1, N_DEV):
            rdma = pltpu.make_async_remote_copy(
                src_ref=sstats,
                dst_ref=rstats.at[k - 1],
                send_sem=send_sems.at[k - 1],
                recv_sem=recv_sems.at[k - 1],
                device_id=((my + k) % N_DEV,),
                device_id_type=pl.DeviceIdType.MESH,
            )
            rdma.start()
            sends.append(rdma)

        for k in range(1, N_DEV):
            recv = pltpu.make_async_remote_copy(
                src_ref=sstats,
                dst_ref=rstats.at[k - 1],
                send_sem=send_sems.at[k - 1],
                recv_sem=recv_sems.at[k - 1],
                device_id=((my - k) % N_DEV,),
                device_id_type=pl.DeviceIdType.MESH,
            )
            recv.wait_recv()

        gm = lm
        for k in range(1, N_DEV):
            gm = jnp.maximum(gm, rstats[k - 1, :, 0:1])
        gs = ls * jnp.exp(lm - gm)
        for k in range(1, N_DEV):
            gs = gs + rstats[k - 1, :, 1:2] * jnp.exp(rstats[k - 1, :, 0:1] - gm)

        scale = jnp.exp(lm - gm) / gs
        out_ref[:, :] = out_ref[:, :] * scale

        for rdma in sends:
            rdma.wait_send()

    return pl.pallas_call(
        body,
        out_shape=jax.ShapeDtypeStruct((m, n), x.dtype),
        in_specs=[pl.BlockSpec(memory_space=pltpu.VMEM)],
        out_specs=pl.BlockSpec(memory_space=pltpu.VMEM),
        scratch_shapes=[
            pltpu.VMEM((m, 2), x.dtype),
            pltpu.VMEM((N_DEV - 1, m, 2), x.dtype),
            pltpu.SemaphoreType.DMA((N_DEV - 1,)),
            pltpu.SemaphoreType.DMA((N_DEV - 1,)),
        ],
        compiler_params=pltpu.CompilerParams(collective_id=0),
    )(x)


# device time: 20124 ns/iter; 1.1129x vs baseline; 1.1129x over previous
import jax
import jax.numpy as jnp
from jax import lax
from jax.experimental import pallas as pl
from jax.experimental.pallas import tpu as pltpu

N_DEV = 4
C = 2


def kernel(x):
    m, n = x.shape
    mh = m // C

    def body(x_ref, out_ref, sstats, rstats, send_sems, recv_sems):
        my = lax.axis_index("i")

        barrier_sem = pltpu.get_barrier_semaphore()
        for k in range(1, N_DEV):
            pl.semaphore_signal(
                barrier_sem,
                inc=1,
                device_id=((my + k) % N_DEV,),
                device_id_type=pl.DeviceIdType.MESH,
            )
        pl.semaphore_wait(barrier_sem, N_DEV - 1)

        sends = []
        for c in range(C):
            xv = x_ref[pl.ds(c * mh, mh), :]
            lm = jnp.max(xv, axis=1, keepdims=True)
            e = jnp.exp(xv - lm)
            ls = jnp.sum(e, axis=1, keepdims=True)
            out_ref[pl.ds(c * mh, mh), :] = e
            sstats[c, :, 0:1] = lm
            sstats[c, :, 1:2] = ls
            for k in range(1, N_DEV):
                rdma = pltpu.make_async_remote_copy(
                    src_ref=sstats.at[c],
                    dst_ref=rstats.at[c, k - 1],
                    send_sem=send_sems.at[c, k - 1],
                    recv_sem=recv_sems.at[c, k - 1],
                    device_id=((my + k) % N_DEV,),
                    device_id_type=pl.DeviceIdType.MESH,
                )
                rdma.start()
                sends.append(rdma)

        for c in range(C):
            for k in range(1, N_DEV):
                recv = pltpu.make_async_remote_copy(
                    src_ref=sstats.at[c],
                    dst_ref=rstats.at[c, k - 1],
                    send_sem=send_sems.at[c, k - 1],
                    recv_sem=recv_sems.at[c, k - 1],
                    device_id=((my - k) % N_DEV,),
                    device_id_type=pl.DeviceIdType.MESH,
                )
                recv.wait_recv()

            lm = sstats[c, :, 0:1]
            ls = sstats[c, :, 1:2]
            gm = lm
            for k in range(1, N_DEV):
                gm = jnp.maximum(gm, rstats[c, k - 1, :, 0:1])
            gs = ls * jnp.exp(lm - gm)
            for k in range(1, N_DEV):
                gs = gs + rstats[c, k - 1, :, 1:2] * jnp.exp(
                    rstats[c, k - 1, :, 0:1] - gm
                )

            scale = jnp.exp(lm - gm) / gs
            sl = pl.ds(c * mh, mh)
            out_ref[sl, :] = out_ref[sl, :] * scale

        for rdma in sends:
            rdma.wait_send()

    return pl.pallas_call(
        body,
        out_shape=jax.ShapeDtypeStruct((m, n), x.dtype),
        in_specs=[pl.BlockSpec(memory_space=pltpu.VMEM)],
        out_specs=pl.BlockSpec(memory_space=pltpu.VMEM),
        scratch_shapes=[
            pltpu.VMEM((C, mh, 2), x.dtype),
            pltpu.VMEM((C, N_DEV - 1, mh, 2), x.dtype),
            pltpu.SemaphoreType.DMA((C, N_DEV - 1)),
            pltpu.SemaphoreType.DMA((C, N_DEV - 1)),
        ],
        compiler_params=pltpu.CompilerParams(collective_id=0),
    )(x)
